# baseline (speedup 1.0000x reference)
import functools

import jax
import jax.numpy as jnp
from jax import lax
from jax.experimental import pallas as pl
from jax.experimental.pallas import tpu as pltpu, tpu_sc as plsc

_NW = 32
_CB = 2000
_L = 16


@functools.cache
def _build(n, m):
    num_chunks = n // _CB
    groups = _CB // _L
    mesh = plsc.VectorSubcoreMesh(core_axis_name="c", subcore_axis_name="s")
    f32 = jnp.float32

    @functools.partial(
        pl.kernel,
        out_type=tuple(jax.ShapeDtypeStruct((n,), f32) for _ in range(6)),
        mesh=mesh,
        scratch_types=[
            pltpu.VMEM((m, 4, 4), f32),
        ] + [pltpu.VMEM((_CB,), f32) for _ in range(6)]
          + [pltpu.VMEM((_CB,), f32) for _ in range(4)]
          + [pltpu.VMEM((_CB,), jnp.int32) for _ in range(4)]
          + [pltpu.VMEM((_CB,), f32) for _ in range(6)]
          + [pltpu.SemaphoreType.DMA],
        compiler_params=pltpu.CompilerParams(
            use_tc_tiling_on_sc=False, needs_layout_passes=False),
    )
    def skin(*refs):
        ins = refs[:15]
        outs = refs[15:21]
        tab_v = refs[21]
        in_v = refs[22:36]
        out_v = refs[36:42]
        sem = refs[42]

        cid = lax.axis_index("c")
        sid = lax.axis_index("s")
        wid = sid * 2 + cid

        pltpu.sync_copy(ins[14], tab_v)

        col = [jnp.full((_L,), d, jnp.int32) for d in range(4)]

        def group(g, carry):
            s = pl.ds(g * _L, _L)
            x, y, z = in_v[0][s], in_v[1][s], in_v[2][s]
            nx, ny, nz = in_v[3][s], in_v[4][s], in_v[5][s]
            av0 = av1 = av2 = jnp.zeros((_L,), f32)
            an0 = an1 = an2 = jnp.zeros((_L,), f32)
            for i in range(4):
                w = in_v[6 + i][s]
                bi = in_v[10 + i][s]
                mm = [plsc.load_gather(tab_v, [bi, col[k // 4], col[k % 4]])
                      for k in range(16)]
                t0 = (x * mm[0] + y * mm[1]) + (z * mm[2] + mm[3])
                t1 = (x * mm[4] + y * mm[5]) + (z * mm[6] + mm[7])
                t2 = (x * mm[8] + y * mm[9]) + (z * mm[10] + mm[11])
                t3 = (x * mm[12] + y * mm[13]) + (z * mm[14] + mm[15])
                r = w / t3
                av0 = av0 + t0 * r
                av1 = av1 + t1 * r
                av2 = av2 + t2 * r
                an0 = an0 + w * (nx * mm[0] + ny * mm[1] + nz * mm[2])
                an1 = an1 + w * (nx * mm[4] + ny * mm[5] + nz * mm[6])
                an2 = an2 + w * (nx * mm[8] + ny * mm[9] + nz * mm[10])
            out_v[0][s], out_v[1][s], out_v[2][s] = av0, av1, av2
            out_v[3][s], out_v[4][s], out_v[5][s] = an0, an1, an2
            return carry

        def chunk(ci, carry):
            c = wid + ci * _NW
            b = c * _CB
            cps = [pltpu.make_async_copy(ins[j].at[pl.ds(b, _CB)], in_v[j], sem)
                   for j in range(14)]
            for cp in cps:
                cp.start()
            for cp in cps:
                cp.wait()
            lax.fori_loop(0, groups, group, 0, unroll=False)
            ocs = [pltpu.make_async_copy(out_v[j], outs[j].at[pl.ds(b, _CB)], sem)
                   for j in range(6)]
            for oc in ocs:
                oc.start()
            for oc in ocs:
                oc.wait()
            return carry

        my_chunks = (num_chunks - 1 - wid) // _NW + 1
        lax.fori_loop(0, my_chunks, chunk, 0, unroll=False)

    return skin


def kernel(vertices, normals, bone_weights, bone_indices, bone_matrices):
    n = vertices.shape[0]
    m = bone_matrices.shape[0]
    pad = (-n) % _CB
    if pad:
        vertices = jnp.pad(vertices, ((0, pad), (0, 0)))
        normals = jnp.pad(normals, ((0, pad), (0, 0)))
        bone_weights = jnp.pad(bone_weights, ((0, pad), (0, 0)))
        bone_indices = jnp.pad(bone_indices, ((0, pad), (0, 0)))
    vertices = vertices.astype(jnp.float32)
    normals = normals.astype(jnp.float32)
    bone_weights = bone_weights.astype(jnp.float32)
    idx32 = bone_indices.astype(jnp.int32)
    planes = (
        [vertices[:, d] for d in range(3)]
        + [normals[:, d] for d in range(3)]
        + [bone_weights[:, d] for d in range(4)]
        + [idx32[:, d] for d in range(4)]
    )
    outs = _build(n + pad, m)(*planes, bone_matrices.astype(jnp.float32))
    ov = jnp.stack(outs[:3], axis=1)[:n]
    on = jnp.stack(outs[3:], axis=1)[:n]
    return ov, on


# speedup vs baseline: 1.0197x; 1.0197x over previous
import functools

import jax
import jax.numpy as jnp
from jax import lax
from jax.experimental import pallas as pl
from jax.experimental.pallas import tpu as pltpu, tpu_sc as plsc

_NW = 32
_CB = 2000
_L = 16


@functools.cache
def _build(n, m):
    num_chunks = n // _CB
    groups = _CB // _L
    mesh = plsc.VectorSubcoreMesh(core_axis_name="c", subcore_axis_name="s")
    f32 = jnp.float32

    @functools.partial(
        pl.kernel,
        out_type=tuple(jax.ShapeDtypeStruct((n,), f32) for _ in range(6)),
        mesh=mesh,
        scratch_types=[
            pltpu.VMEM((m, 4, 4), f32),
        ] + [pltpu.VMEM((_CB,), f32) for _ in range(6)]
          + [pltpu.VMEM((_CB,), f32) for _ in range(4)]
          + [pltpu.VMEM((_CB,), jnp.int32) for _ in range(4)]
          + [pltpu.VMEM((_CB,), f32) for _ in range(6)]
          + [pltpu.SemaphoreType.DMA],
        compiler_params=pltpu.CompilerParams(
            use_tc_tiling_on_sc=False, needs_layout_passes=False),
    )
    def skin(*refs):
        ins = refs[:15]
        outs = refs[15:21]
        tab_v = refs[21]
        in_v = refs[22:36]
        out_v = refs[36:42]
        sem = refs[42]

        cid = lax.axis_index("c")
        sid = lax.axis_index("s")
        wid = sid * 2 + cid

        pltpu.sync_copy(ins[14], tab_v)

        col = [jnp.full((_L,), d, jnp.int32) for d in range(4)]

        def group(g):
            s = pl.ds(g * _L, _L)
            x, y, z = in_v[0][s], in_v[1][s], in_v[2][s]
            nx, ny, nz = in_v[3][s], in_v[4][s], in_v[5][s]
            av0 = av1 = av2 = jnp.zeros((_L,), f32)
            an0 = an1 = an2 = jnp.zeros((_L,), f32)
            for i in range(4):
                w = in_v[6 + i][s]
                bi = in_v[10 + i][s]
                mm = [plsc.load_gather(tab_v, [bi, col[k // 4], col[k % 4]])
                      for k in range(16)]
                t0 = (x * mm[0] + y * mm[1]) + (z * mm[2] + mm[3])
                t1 = (x * mm[4] + y * mm[5]) + (z * mm[6] + mm[7])
                t2 = (x * mm[8] + y * mm[9]) + (z * mm[10] + mm[11])
                t3 = (x * mm[12] + y * mm[13]) + (z * mm[14] + mm[15])
                r = w / t3
                av0 = av0 + t0 * r
                av1 = av1 + t1 * r
                av2 = av2 + t2 * r
                an0 = an0 + w * (nx * mm[0] + ny * mm[1] + nz * mm[2])
                an1 = an1 + w * (nx * mm[4] + ny * mm[5] + nz * mm[6])
                an2 = an2 + w * (nx * mm[8] + ny * mm[9] + nz * mm[10])
            out_v[0][s], out_v[1][s], out_v[2][s] = av0, av1, av2
            out_v[3][s], out_v[4][s], out_v[5][s] = an0, an1, an2

        def chunk(ci, carry):
            c = wid + ci * _NW
            b = c * _CB
            cps = [pltpu.make_async_copy(ins[j].at[pl.ds(b, _CB)], in_v[j], sem)
                   for j in range(14)]
            for cp in cps:
                cp.start()
            for cp in cps:
                cp.wait()
            plsc.parallel_loop(0, groups, 1, unroll=4)(group)
            ocs = [pltpu.make_async_copy(out_v[j], outs[j].at[pl.ds(b, _CB)], sem)
                   for j in range(6)]
            for oc in ocs:
                oc.start()
            for oc in ocs:
                oc.wait()
            return carry

        my_chunks = (num_chunks - 1 - wid) // _NW + 1
        lax.fori_loop(0, my_chunks, chunk, 0, unroll=False)

    return skin


def kernel(vertices, normals, bone_weights, bone_indices, bone_matrices):
    n = vertices.shape[0]
    m = bone_matrices.shape[0]
    pad = (-n) % _CB
    if pad:
        vertices = jnp.pad(vertices, ((0, pad), (0, 0)))
        normals = jnp.pad(normals, ((0, pad), (0, 0)))
        bone_weights = jnp.pad(bone_weights, ((0, pad), (0, 0)))
        bone_indices = jnp.pad(bone_indices, ((0, pad), (0, 0)))
    vertices = vertices.astype(jnp.float32)
    normals = normals.astype(jnp.float32)
    bone_weights = bone_weights.astype(jnp.float32)
    idx32 = bone_indices.astype(jnp.int32)
    planes = (
        [vertices[:, d] for d in range(3)]
        + [normals[:, d] for d in range(3)]
        + [bone_weights[:, d] for d in range(4)]
        + [idx32[:, d] for d in range(4)]
    )
    outs = _build(n + pad, m)(*planes, bone_matrices.astype(jnp.float32))
    ov = jnp.stack(outs[:3], axis=1)[:n]
    on = jnp.stack(outs[3:], axis=1)[:n]
    return ov, on


# speedup vs baseline: 2.1221x; 2.0811x over previous
import functools

import jax
import jax.numpy as jnp
from jax import lax
from jax.experimental import pallas as pl
from jax.experimental.pallas import tpu as pltpu, tpu_sc as plsc

_NW = 32
_CB = 2000
_L = 16


@functools.cache
def _build(n, m):
    num_chunks = n // _CB
    groups = _CB // _L
    mesh = plsc.VectorSubcoreMesh(core_axis_name="c", subcore_axis_name="s")
    f32 = jnp.float32

    @functools.partial(
        pl.kernel,
        out_type=tuple(jax.ShapeDtypeStruct((n,), f32) for _ in range(6)),
        mesh=mesh,
        scratch_types=[
            pltpu.VMEM((m * 16,), f32),
        ] + [pltpu.VMEM((_CB,), f32) for _ in range(6)]
          + [pltpu.VMEM((_CB,), f32) for _ in range(4)]
          + [pltpu.VMEM((_CB,), jnp.int32) for _ in range(4)]
          + [pltpu.VMEM((_CB,), f32) for _ in range(6)]
          + [pltpu.SemaphoreType.DMA],
        compiler_params=pltpu.CompilerParams(
            use_tc_tiling_on_sc=False, needs_layout_passes=False),
    )
    def skin(*refs):
        ins = refs[:15]
        outs = refs[15:21]
        tab_v = refs[21]
        in_v = refs[22:36]
        out_v = refs[36:42]
        sem = refs[42]

        cid = lax.axis_index("c")
        sid = lax.axis_index("s")
        wid = sid * 2 + cid

        pltpu.sync_copy(ins[14], tab_v)

        def group(g):
            s = pl.ds(g * _L, _L)
            x, y, z = in_v[0][s], in_v[1][s], in_v[2][s]
            nx, ny, nz = in_v[3][s], in_v[4][s], in_v[5][s]
            av0 = av1 = av2 = jnp.zeros((_L,), f32)
            an0 = an1 = an2 = jnp.zeros((_L,), f32)
            for i in range(4):
                w = in_v[6 + i][s]
                bi = in_v[10 + i][s]
                mm = [plsc.load_gather(tab_v, [bi + (k * m)])
                      for k in range(16)]
                t0 = (x * mm[0] + y * mm[1]) + (z * mm[2] + mm[3])
                t1 = (x * mm[4] + y * mm[5]) + (z * mm[6] + mm[7])
                t2 = (x * mm[8] + y * mm[9]) + (z * mm[10] + mm[11])
                t3 = (x * mm[12] + y * mm[13]) + (z * mm[14] + mm[15])
                r = w / t3
                av0 = av0 + t0 * r
                av1 = av1 + t1 * r
                av2 = av2 + t2 * r
                an0 = an0 + w * (nx * mm[0] + ny * mm[1] + nz * mm[2])
                an1 = an1 + w * (nx * mm[4] + ny * mm[5] + nz * mm[6])
                an2 = an2 + w * (nx * mm[8] + ny * mm[9] + nz * mm[10])
            out_v[0][s], out_v[1][s], out_v[2][s] = av0, av1, av2
            out_v[3][s], out_v[4][s], out_v[5][s] = an0, an1, an2

        def chunk(ci, carry):
            c = wid + ci * _NW
            b = c * _CB
            cps = [pltpu.make_async_copy(ins[j].at[pl.ds(b, _CB)], in_v[j], sem)
                   for j in range(14)]
            for cp in cps:
                cp.start()
            for cp in cps:
                cp.wait()
            plsc.parallel_loop(0, groups, 1, unroll=4)(group)
            ocs = [pltpu.make_async_copy(out_v[j], outs[j].at[pl.ds(b, _CB)], sem)
                   for j in range(6)]
            for oc in ocs:
                oc.start()
            for oc in ocs:
                oc.wait()
            return carry

        my_chunks = (num_chunks - 1 - wid) // _NW + 1
        lax.fori_loop(0, my_chunks, chunk, 0, unroll=False)

    return skin


def kernel(vertices, normals, bone_weights, bone_indices, bone_matrices):
    n = vertices.shape[0]
    m = bone_matrices.shape[0]
    pad = (-n) % _CB
    if pad:
        vertices = jnp.pad(vertices, ((0, pad), (0, 0)))
        normals = jnp.pad(normals, ((0, pad), (0, 0)))
        bone_weights = jnp.pad(bone_weights, ((0, pad), (0, 0)))
        bone_indices = jnp.pad(bone_indices, ((0, pad), (0, 0)))
    vertices = vertices.astype(jnp.float32)
    normals = normals.astype(jnp.float32)
    bone_weights = bone_weights.astype(jnp.float32)
    idx32 = bone_indices.astype(jnp.int32)
    planes = (
        [vertices[:, d] for d in range(3)]
        + [normals[:, d] for d in range(3)]
        + [bone_weights[:, d] for d in range(4)]
        + [idx32[:, d] for d in range(4)]
    )
    tab_t = bone_matrices.astype(jnp.float32).reshape(m, 16).T.reshape(-1)
    outs = _build(n + pad, m)(*planes, tab_t)
    ov = jnp.stack(outs[:3], axis=1)[:n]
    on = jnp.stack(outs[3:], axis=1)[:n]
    return ov, on


# speedup vs baseline: 2.2828x; 1.0757x over previous
import functools

import jax
import jax.numpy as jnp
from jax import lax
from jax.experimental import pallas as pl
from jax.experimental.pallas import tpu as pltpu, tpu_sc as plsc

_NW = 32
_CB = 2000
_L = 16
_NPL = 11


@functools.cache
def _build(n, m):
    num_chunks = n // _CB
    groups = _CB // _L
    max_my = (num_chunks + _NW - 1) // _NW
    assert max_my % 2 == 0, "pipeline assumes an even per-worker chunk bound"
    mesh = plsc.VectorSubcoreMesh(core_axis_name="c", subcore_axis_name="s")
    f32 = jnp.float32

    @functools.partial(
        pl.kernel,
        out_type=tuple(jax.ShapeDtypeStruct((n,), f32) for _ in range(6)),
        mesh=mesh,
        scratch_types=[pltpu.VMEM((m * 16,), f32)]
        + [pltpu.VMEM((_CB,), jnp.int32 if p == _NPL - 1 else f32)
           for _ in range(2) for p in range(_NPL)]
        + [pltpu.VMEM((_CB,), f32) for _ in range(12)]
        + [pltpu.SemaphoreType.DMA for _ in range(4)],
        compiler_params=pltpu.CompilerParams(
            use_tc_tiling_on_sc=False, needs_layout_passes=False),
    )
    def skin(*refs):
        ins = refs[:12]
        outs = refs[12:18]
        tab_v = refs[18]
        in_v = (refs[19:19 + _NPL], refs[19 + _NPL:19 + 2 * _NPL])
        out_v = (refs[41:47], refs[47:53])
        in_sem = refs[53:55]
        out_sem = refs[55:57]

        cid = lax.axis_index("c")
        sid = lax.axis_index("s")
        wid = sid * 2 + cid
        my_chunks = (num_chunks - 1 - wid) // _NW + 1

        pltpu.sync_copy(ins[11], tab_v)

        def in_copies(ci, b):
            base = (wid + ci * _NW) * _CB
            return [pltpu.make_async_copy(
                ins[j].at[pl.ds(base, _CB)], in_v[b][j], in_sem[b])
                for j in range(_NPL)]

        def out_copies(ci, b):
            base = (wid + ci * _NW) * _CB
            return [pltpu.make_async_copy(
                out_v[b][j], outs[j].at[pl.ds(base, _CB)], out_sem[b])
                for j in range(6)]

        def compute(b):
            iv, ov = in_v[b], out_v[b]

            def group(g):
                s = pl.ds(g * _L, _L)
                x, y, z = iv[0][s], iv[1][s], iv[2][s]
                nx, ny, nz = iv[3][s], iv[4][s], iv[5][s]
                bp = iv[10][s]
                av0 = av1 = av2 = jnp.zeros((_L,), f32)
                an0 = an1 = an2 = jnp.zeros((_L,), f32)
                for i in range(4):
                    w = iv[6 + i][s]
                    bi = lax.shift_right_logical(bp, 8 * i) & 0xFF
                    mm = [plsc.load_gather(tab_v, [bi + (k * m)])
                          for k in range(16)]
                    t0 = (x * mm[0] + y * mm[1]) + (z * mm[2] + mm[3])
                    t1 = (x * mm[4] + y * mm[5]) + (z * mm[6] + mm[7])
                    t2 = (x * mm[8] + y * mm[9]) + (z * mm[10] + mm[11])
                    t3 = (x * mm[12] + y * mm[13]) + (z * mm[14] + mm[15])
                    r = w / t3
                    av0 = av0 + t0 * r
                    av1 = av1 + t1 * r
                    av2 = av2 + t2 * r
                    an0 = an0 + w * (nx * mm[0] + ny * mm[1] + nz * mm[2])
                    an1 = an1 + w * (nx * mm[4] + ny * mm[5] + nz * mm[6])
                    an2 = an2 + w * (nx * mm[8] + ny * mm[9] + nz * mm[10])
                ov[0][s], ov[1][s], ov[2][s] = av0, av1, av2
                ov[3][s], ov[4][s], ov[5][s] = an0, an1, an2

            plsc.parallel_loop(0, groups, 1, unroll=4)(group)

        for cp in in_copies(0, 0):
            cp.start()

        def pipe(i, carry):
            for b in range(2):
                ci = 2 * i + b
                active = ci < my_chunks

                @pl.when(ci + 1 < my_chunks)
                def _():
                    for cp in in_copies(ci + 1, 1 - b):
                        cp.start()

                @pl.when(active)
                def _():
                    for cp in in_copies(ci, b):
                        cp.wait()

                @pl.when(active & (ci >= 2))
                def _():
                    for oc in out_copies(ci - 2, b):
                        oc.wait()

                @pl.when(active)
                def _():
                    compute(b)
                    for oc in out_copies(ci, b):
                        oc.start()

            return carry

        lax.fori_loop(0, max_my // 2, pipe, 0, unroll=False)

        for b in range(2):
            for j in range(6):
                pltpu.make_async_copy(
                    out_v[b][j], outs[j].at[pl.ds(0, _CB)], out_sem[b]).wait()

    return skin


def kernel(vertices, normals, bone_weights, bone_indices, bone_matrices):
    n = vertices.shape[0]
    m = bone_matrices.shape[0]
    pad = (-n) % _CB
    if pad:
        vertices = jnp.pad(vertices, ((0, pad), (0, 0)))
        normals = jnp.pad(normals, ((0, pad), (0, 0)))
        bone_weights = jnp.pad(bone_weights, ((0, pad), (0, 0)))
        bone_indices = jnp.pad(bone_indices, ((0, pad), (0, 0)))
    vertices = vertices.astype(jnp.float32)
    normals = normals.astype(jnp.float32)
    bone_weights = bone_weights.astype(jnp.float32)
    idx32 = bone_indices.astype(jnp.int32)
    packed = (idx32[:, 0] | (idx32[:, 1] << 8) | (idx32[:, 2] << 16)
              | (idx32[:, 3] << 24))
    planes = (
        [vertices[:, d] for d in range(3)]
        + [normals[:, d] for d in range(3)]
        + [bone_weights[:, d] for d in range(4)]
        + [packed]
    )
    tab_t = bone_matrices.astype(jnp.float32).reshape(m, 16).T.reshape(-1)
    outs = _build(n + pad, m)(*planes, tab_t)
    ov = jnp.stack(outs[:3], axis=1)[:n]
    on = jnp.stack(outs[3:], axis=1)[:n]
    return ov, on


# speedup vs baseline: 2.3534x; 1.0309x over previous
import functools

import jax
import jax.numpy as jnp
from jax import lax
from jax.experimental import pallas as pl
from jax.experimental.pallas import tpu as pltpu, tpu_sc as plsc

_NW = 32
_CB = 2000
_L = 16
_NPL = 11


@functools.cache
def _build(n, m):
    num_chunks = n // _CB
    groups = _CB // _L
    max_my = (num_chunks + _NW - 1) // _NW
    assert max_my % 2 == 0, "pipeline assumes an even per-worker chunk bound"
    mesh = plsc.VectorSubcoreMesh(core_axis_name="c", subcore_axis_name="s")
    f32 = jnp.float32

    @functools.partial(
        pl.kernel,
        out_type=tuple(jax.ShapeDtypeStruct((n,), f32) for _ in range(6)),
        mesh=mesh,
        scratch_types=[pltpu.VMEM((m * 16,), f32)]
        + [pltpu.VMEM((_CB,), jnp.int32 if p == _NPL - 1 else f32)
           for _ in range(2) for p in range(_NPL)]
        + [pltpu.VMEM((_CB,), f32) for _ in range(12)]
        + [pltpu.SemaphoreType.DMA for _ in range(4)],
        compiler_params=pltpu.CompilerParams(
            use_tc_tiling_on_sc=False, needs_layout_passes=False),
    )
    def skin(*refs):
        ins = refs[:12]
        outs = refs[12:18]
        tab_v = refs[18]
        in_v = (refs[19:19 + _NPL], refs[19 + _NPL:19 + 2 * _NPL])
        out_v = (refs[41:47], refs[47:53])
        in_sem = refs[53:55]
        out_sem = refs[55:57]

        cid = lax.axis_index("c")
        sid = lax.axis_index("s")
        wid = sid * 2 + cid
        my_chunks = (num_chunks - 1 - wid) // _NW + 1

        pltpu.sync_copy(ins[11], tab_v)

        def in_copies(ci, b):
            base = (wid + ci * _NW) * _CB
            return [pltpu.make_async_copy(
                ins[j].at[pl.ds(base, _CB)], in_v[b][j], in_sem[b])
                for j in range(_NPL)]

        def out_copies(ci, b):
            base = (wid + ci * _NW) * _CB
            return [pltpu.make_async_copy(
                out_v[b][j], outs[j].at[pl.ds(base, _CB)], out_sem[b])
                for j in range(6)]

        def compute(b):
            iv, ov = in_v[b], out_v[b]

            def group(g):
                s = pl.ds(g * _L, _L)
                x, y, z = iv[0][s], iv[1][s], iv[2][s]
                nx, ny, nz = iv[3][s], iv[4][s], iv[5][s]
                bp = iv[10][s]
                av0 = av1 = av2 = jnp.zeros((_L,), f32)
                an0 = an1 = an2 = jnp.zeros((_L,), f32)
                for i in range(4):
                    w = iv[6 + i][s]
                    bi = lax.shift_right_logical(bp, 8 * i) & 0xFF
                    mm = [plsc.load_gather(tab_v.at[pl.ds(k * m, m)], [bi])
                          for k in range(16)]
                    t0 = (x * mm[0] + y * mm[1]) + (z * mm[2] + mm[3])
                    t1 = (x * mm[4] + y * mm[5]) + (z * mm[6] + mm[7])
                    t2 = (x * mm[8] + y * mm[9]) + (z * mm[10] + mm[11])
                    t3 = (x * mm[12] + y * mm[13]) + (z * mm[14] + mm[15])
                    r = w / t3
                    av0 = av0 + t0 * r
                    av1 = av1 + t1 * r
                    av2 = av2 + t2 * r
                    an0 = an0 + w * (nx * mm[0] + ny * mm[1] + nz * mm[2])
                    an1 = an1 + w * (nx * mm[4] + ny * mm[5] + nz * mm[6])
                    an2 = an2 + w * (nx * mm[8] + ny * mm[9] + nz * mm[10])
                ov[0][s], ov[1][s], ov[2][s] = av0, av1, av2
                ov[3][s], ov[4][s], ov[5][s] = an0, an1, an2

            plsc.parallel_loop(0, groups, 1, unroll=4)(group)

        for cp in in_copies(0, 0):
            cp.start()

        def pipe(i, carry):
            for b in range(2):
                ci = 2 * i + b
                active = ci < my_chunks

                @pl.when(ci + 1 < my_chunks)
                def _():
                    for cp in in_copies(ci + 1, 1 - b):
                        cp.start()

                @pl.when(active)
                def _():
                    for cp in in_copies(ci, b):
                        cp.wait()

                @pl.when(active & (ci >= 2))
                def _():
                    for oc in out_copies(ci - 2, b):
                        oc.wait()

                @pl.when(active)
                def _():
                    compute(b)
                    for oc in out_copies(ci, b):
                        oc.start()

            return carry

        lax.fori_loop(0, max_my // 2, pipe, 0, unroll=False)

        for b in range(2):
            for j in range(6):
                pltpu.make_async_copy(
                    out_v[b][j], outs[j].at[pl.ds(0, _CB)], out_sem[b]).wait()

    return skin


def kernel(vertices, normals, bone_weights, bone_indices, bone_matrices):
    n = vertices.shape[0]
    m = bone_matrices.shape[0]
    pad = (-n) % _CB
    if pad:
        vertices = jnp.pad(vertices, ((0, pad), (0, 0)))
        normals = jnp.pad(normals, ((0, pad), (0, 0)))
        bone_weights = jnp.pad(bone_weights, ((0, pad), (0, 0)))
        bone_indices = jnp.pad(bone_indices, ((0, pad), (0, 0)))
    vertices = vertices.astype(jnp.float32)
    normals = normals.astype(jnp.float32)
    bone_weights = bone_weights.astype(jnp.float32)
    idx32 = bone_indices.astype(jnp.int32)
    packed = (idx32[:, 0] | (idx32[:, 1] << 8) | (idx32[:, 2] << 16)
              | (idx32[:, 3] << 24))
    planes = (
        [vertices[:, d] for d in range(3)]
        + [normals[:, d] for d in range(3)]
        + [bone_weights[:, d] for d in range(4)]
        + [packed]
    )
    tab_t = bone_matrices.astype(jnp.float32).reshape(m, 16).T.reshape(-1)
    outs = _build(n + pad, m)(*planes, tab_t)
    ov = jnp.stack(outs[:3], axis=1)[:n]
    on = jnp.stack(outs[3:], axis=1)[:n]
    return ov, on
